# Initial kernel scaffold; baseline (speedup 1.0000x reference)
import jax
import jax.numpy as jnp
from jax.experimental import pallas as pl


def kernel(x, W, b):
    raise NotImplementedError("write your pallas kernel here")


# speedup vs baseline: 1.9942x; 1.9942x over previous
import functools

import jax
import jax.numpy as jnp
from jax import lax
from jax.experimental import pallas as pl
from jax.experimental.pallas import tpu as pltpu
from jax.experimental.pallas import tpu_sc as plsc

N_EXPERTS = 8
TOP_K = 2
N_TOKENS = 32768
D_MODEL = 768

_NC = 2
_NS = 16
_NW = _NC * _NS
_TOK_PER_TILE = N_TOKENS // _NW
_GROUPS = _TOK_PER_TILE // 16


def _scores_kernel(x_ref, w_ref, s_ref):
    x = x_ref[...]
    w = w_ref[...]
    k2 = D_MODEL // 2
    a = lax.dot_general(w[:, :k2], x[:, :k2], (((1,), (1,)), ((), ())),
                        preferred_element_type=jnp.float32)
    c = lax.dot_general(w[:, k2:], x[:, k2:], (((1,), (1,)), ((), ())),
                        preferred_element_type=jnp.float32)
    s_ref[...] = a + c


def _router_body(s_hbm, b_hbm, wout_hbm, iout_hbm, sbuf, bbuf, wbuf, ibuf):
    wid = lax.axis_index("s") * _NC + lax.axis_index("c")
    base = wid * _TOK_PER_TILE
    pltpu.sync_copy(s_hbm.at[:, pl.ds(base, _TOK_PER_TILE)], sbuf)
    pltpu.sync_copy(b_hbm, bbuf)
    iota = lax.iota(jnp.int32, 16)

    def group(g, carry):
        off = g * 16
        s = [sbuf[e, pl.ds(off, 16)] for e in range(N_EXPERTS)]
        mx = s[0]
        for e in range(1, N_EXPERTS):
            mx = jnp.maximum(mx, s[e])
        ex = [jnp.exp(v - mx) for v in s]
        den = ex[0]
        for e in range(1, N_EXPERTS):
            den = den + ex[e]
        p = [v / den for v in ex]
        sb = [p[e] + bbuf[e] for e in range(N_EXPERTS)]
        m1 = sb[0]
        i1 = jnp.zeros(16, jnp.int32)
        for e in range(1, N_EXPERTS):
            upd = sb[e] > m1
            m1 = jnp.where(upd, sb[e], m1)
            i1 = jnp.where(upd, e, i1)
        m2 = jnp.full(16, -jnp.inf, jnp.float32)
        i2 = jnp.zeros(16, jnp.int32)
        for e in range(N_EXPERTS):
            upd = (i1 != e) & (sb[e] > m2)
            m2 = jnp.where(upd, sb[e], m2)
            i2 = jnp.where(upd, e, i2)
        w1 = p[0]
        w2 = p[0]
        for e in range(1, N_EXPERTS):
            w1 = jnp.where(i1 == e, p[e], w1)
            w2 = jnp.where(i2 == e, p[e], w2)
        wbuf[0, pl.ds(off, 16)] = w1
        wbuf[1, pl.ds(off, 16)] = w2
        ibuf[0, pl.ds(off, 16)] = i1
        ibuf[1, pl.ds(off, 16)] = i2
        return carry

    lax.fori_loop(0, _GROUPS, group, 0)
    pltpu.sync_copy(wbuf, wout_hbm.at[:, pl.ds(base, _TOK_PER_TILE)])
    pltpu.sync_copy(ibuf, iout_hbm.at[:, pl.ds(base, _TOK_PER_TILE)])


@jax.jit
def kernel(x, W, b):
    n_tokens, d_model = x.shape
    block = 4096
    scores_t = pl.pallas_call(
        _scores_kernel,
        grid=(n_tokens // block,),
        in_specs=[
            pl.BlockSpec((block, d_model), lambda i: (i, 0)),
            pl.BlockSpec((N_EXPERTS, d_model), lambda i: (0, 0)),
        ],
        out_specs=pl.BlockSpec((N_EXPERTS, block), lambda i: (0, i)),
        out_shape=jax.ShapeDtypeStruct((N_EXPERTS, n_tokens), jnp.float32),
    )(x, W)

    b_tiled = jnp.broadcast_to(b.reshape(N_EXPERTS, 1), (N_EXPERTS, 16))

    router = pl.kernel(
        _router_body,
        out_type=[
            jax.ShapeDtypeStruct((TOP_K, n_tokens), jnp.float32),
            jax.ShapeDtypeStruct((TOP_K, n_tokens), jnp.int32),
        ],
        mesh=plsc.VectorSubcoreMesh(core_axis_name="c", subcore_axis_name="s"),
        scratch_types=[
            pltpu.VMEM((N_EXPERTS, _TOK_PER_TILE), jnp.float32),
            pltpu.VMEM((N_EXPERTS, 16), jnp.float32),
            pltpu.VMEM((TOP_K, _TOK_PER_TILE), jnp.float32),
            pltpu.VMEM((TOP_K, _TOK_PER_TILE), jnp.int32),
        ],
    )
    w_t, i_t = router(scores_t, b_tiled)
    return w_t.T, i_t.T
